# baseline (speedup 1.0000x reference)
import functools

import jax
import jax.numpy as jnp
import numpy as np
from jax.experimental import pallas as pl
from jax.experimental.pallas import tpu as pltpu

_BN = 10000
_WSZ = 64
_NWIN = 16


def _pool_body(nf_ref, seg_ref, deg_ref, g_ref, awr_ref, w1t_ref,
               w2t_ref, b1_ref, b2_ref, out_ref, sd_acc, gz_ref,
               gzv_ref, *, nblocks, b_real):
    i = pl.program_id(0)
    h = nf_ref.shape[1]

    @pl.when(i == 0)
    def _init():
        sd_acc[...] = jnp.zeros_like(sd_acc)
        gz_ref[...] = jnp.zeros_like(gz_ref)
        gz_ref[pl.ds(0, b_real), :] = jax.lax.dot(
            g_ref[...], awr_ref[...], preferred_element_type=jnp.float32)

    nf = nf_ref[...]
    segf = seg_ref[...]
    deg = deg_ref[...]
    bn = nf.shape[0]

    zraw = jax.lax.dot(nf, awr_ref[...],
                       preferred_element_type=jnp.float32)

    s_first = segf[0, 0]
    s_last = segf[bn - 1, 0]
    m_first = jnp.floor(s_first / _WSZ)
    m_last = jnp.floor(s_last / _WSZ)

    col = jax.lax.broadcasted_iota(jnp.int32, (bn, _WSZ), 1).astype(jnp.float32)

    gzv_ref[...] = jnp.zeros_like(gzv_ref)
    for w in range(_NWIN):
        m = m_first + w

        @pl.when(m <= m_last)
        def _gather(m=m):
            onehot = (segf - m * _WSZ == col).astype(jnp.float32)
            base = m.astype(jnp.int32) * _WSZ
            gzv_ref[...] += jax.lax.dot(
                onehot, gz_ref[pl.ds(base, _WSZ), :],
                preferred_element_type=jnp.float32)

    w128 = jnp.exp((zraw + gzv_ref[...]) * deg)
    u2 = jnp.concatenate([nf * w128, w128], axis=1)

    for w in range(_NWIN):
        m = m_first + w

        @pl.when(m <= m_last)
        def _scatter(m=m):
            onehot = (segf - m * _WSZ == col).astype(jnp.float32)
            base = m.astype(jnp.int32) * _WSZ
            sd_acc[pl.ds(base, _WSZ), :] += jax.lax.dot_general(
                onehot, u2, (((0,), (0,)), ((), ())),
                preferred_element_type=jnp.float32)

    @pl.when(i == nblocks - 1)
    def _finish():
        s = sd_acc[pl.ds(0, b_real), pl.ds(0, h)]
        d = sd_acc[pl.ds(0, b_real), pl.ds(h, h)]
        g = g_ref[...]
        he = jnp.where(d > 0.0, s / d + g, 0.0)
        h1 = jax.nn.relu(
            jax.lax.dot(he, w1t_ref[...], preferred_element_type=jnp.float32)
            + b1_ref[...])
        h2 = jax.lax.dot(h1, w2t_ref[...],
                         preferred_element_type=jnp.float32) + b2_ref[...]
        out_ref[...] = h2 + g


def kernel(node_feats, g_feats, degree, segment_ids, attn, W1, b1, W2, b2):
    n, h = node_feats.shape
    b, _ = g_feats.shape
    nh, dh = attn.shape[1], attn.shape[2]

    segf = segment_ids.astype(jnp.float32).reshape(n, 1)

    headmask = np.kron(np.eye(nh, dtype=np.float32),
                       np.ones((dh, dh), np.float32))
    awr = attn.reshape(nh * dh, 1) * headmask

    nblocks = n // _BN
    b_pad = _NWIN * _WSZ

    body = functools.partial(_pool_body, nblocks=nblocks, b_real=b)
    out = pl.pallas_call(
        body,
        grid=(nblocks,),
        in_specs=[
            pl.BlockSpec((_BN, h), lambda i: (i, 0)),
            pl.BlockSpec((_BN, 1), lambda i: (i, 0)),
            pl.BlockSpec((_BN, 1), lambda i: (i, 0)),
            pl.BlockSpec((b, h), lambda i: (0, 0)),
            pl.BlockSpec((h, h), lambda i: (0, 0)),
            pl.BlockSpec((h, h), lambda i: (0, 0)),
            pl.BlockSpec((h, h), lambda i: (0, 0)),
            pl.BlockSpec((1, h), lambda i: (0, 0)),
            pl.BlockSpec((1, h), lambda i: (0, 0)),
        ],
        out_specs=pl.BlockSpec((b, h), lambda i: (0, 0)),
        out_shape=jax.ShapeDtypeStruct((b, h), jnp.float32),
        scratch_shapes=[
            pltpu.VMEM((b_pad, 2 * h), jnp.float32),
            pltpu.VMEM((b_pad, h), jnp.float32),
            pltpu.VMEM((_BN, h), jnp.float32),
        ],
        compiler_params=pltpu.CompilerParams(
            dimension_semantics=("arbitrary",)),
    )(node_feats, segf, degree, g_feats, awr, W1.T, W2.T,
      b1.reshape(1, h), b2.reshape(1, h))
    return out


# speedup vs baseline: 3.1803x; 3.1803x over previous
import functools

import jax
import jax.numpy as jnp
import numpy as np
from jax.experimental import pallas as pl
from jax.experimental.pallas import tpu as pltpu

_BN = 4000
_WSZ = 256
_NWIN = 4


def _pool_body(nf_ref, seg_ref, deg_ref, g_ref, aw_ref, r_ref, w1t_ref,
               w2t_ref, b1_ref, b2_ref, out_ref, sd_acc, gz_ref,
               gzv_ref, *, nblocks, b_real):
    i = pl.program_id(0)
    h = nf_ref.shape[1]

    @pl.when(i == 0)
    def _init():
        sd_acc[...] = jnp.zeros_like(sd_acc)
        gz_ref[...] = jnp.zeros_like(gz_ref)
        gz_ref[pl.ds(0, b_real), :] = jax.lax.dot(
            g_ref[...], aw_ref[...], preferred_element_type=jnp.float32)

    nf = nf_ref[...]
    segf = seg_ref[...]
    deg = deg_ref[...]
    bn = nf.shape[0]

    zraw = jax.lax.dot(nf, aw_ref[...],
                       preferred_element_type=jnp.float32)

    s_first = segf[0, 0]
    s_last = segf[bn - 1, 0]
    m_first = jnp.floor(s_first / _WSZ)
    m_last = jnp.floor(s_last / _WSZ)

    col = jax.lax.broadcasted_iota(jnp.int32, (bn, _WSZ), 1).astype(jnp.float32)

    gzv_ref[...] = jnp.zeros_like(gzv_ref)
    for w in range(_NWIN):
        m = m_first + w

        @pl.when(m <= m_last)
        def _gather(m=m):
            onehot = (segf - m * _WSZ == col).astype(jnp.float32)
            base = m.astype(jnp.int32) * _WSZ
            gzv_ref[...] += jax.lax.dot(
                onehot, gz_ref[pl.ds(base, _WSZ), :],
                preferred_element_type=jnp.float32)

    wexp = jnp.exp((zraw + gzv_ref[...]) * deg)
    w128 = jax.lax.dot(wexp, r_ref[...], preferred_element_type=jnp.float32)
    u2 = jnp.concatenate([nf * w128, w128], axis=1)

    for w in range(_NWIN):
        m = m_first + w

        @pl.when(m <= m_last)
        def _scatter(m=m):
            onehot = (segf - m * _WSZ == col).astype(jnp.float32)
            base = m.astype(jnp.int32) * _WSZ
            sd_acc[pl.ds(base, _WSZ), :] += jax.lax.dot_general(
                onehot, u2, (((0,), (0,)), ((), ())),
                preferred_element_type=jnp.float32)

    @pl.when(i == nblocks - 1)
    def _finish():
        s = sd_acc[pl.ds(0, b_real), pl.ds(0, h)]
        d = sd_acc[pl.ds(0, b_real), pl.ds(h, h)]
        g = g_ref[...]
        he = jnp.where(d > 0.0, s / d + g, 0.0)
        h1 = jax.nn.relu(
            jax.lax.dot(he, w1t_ref[...], preferred_element_type=jnp.float32)
            + b1_ref[...])
        h2 = jax.lax.dot(h1, w2t_ref[...],
                         preferred_element_type=jnp.float32) + b2_ref[...]
        out_ref[...] = h2 + g


def kernel(node_feats, g_feats, degree, segment_ids, attn, W1, b1, W2, b2):
    n, h = node_feats.shape
    b, _ = g_feats.shape
    nh, dh = attn.shape[1], attn.shape[2]

    segf = segment_ids.astype(jnp.float32).reshape(n, 1)

    eye = np.kron(np.eye(nh, dtype=np.float32), np.ones((dh, 1), np.float32))
    aw = jnp.pad(attn.reshape(nh * dh, 1) * eye, ((0, 0), (0, 8 - nh)))
    r = jnp.pad(
        jnp.asarray(np.kron(np.eye(nh, dtype=np.float32),
                            np.ones((1, dh), np.float32))),
        ((0, 8 - nh), (0, 0)))

    nblocks = n // _BN
    b_pad = _NWIN * _WSZ

    body = functools.partial(_pool_body, nblocks=nblocks, b_real=b)
    out = pl.pallas_call(
        body,
        grid=(nblocks,),
        in_specs=[
            pl.BlockSpec((_BN, h), lambda i: (i, 0)),
            pl.BlockSpec((_BN, 1), lambda i: (i, 0)),
            pl.BlockSpec((_BN, 1), lambda i: (i, 0)),
            pl.BlockSpec((b, h), lambda i: (0, 0)),
            pl.BlockSpec((h, 8), lambda i: (0, 0)),
            pl.BlockSpec((8, h), lambda i: (0, 0)),
            pl.BlockSpec((h, h), lambda i: (0, 0)),
            pl.BlockSpec((h, h), lambda i: (0, 0)),
            pl.BlockSpec((1, h), lambda i: (0, 0)),
            pl.BlockSpec((1, h), lambda i: (0, 0)),
        ],
        out_specs=pl.BlockSpec((b, h), lambda i: (0, 0)),
        out_shape=jax.ShapeDtypeStruct((b, h), jnp.float32),
        scratch_shapes=[
            pltpu.VMEM((b_pad, 2 * h), jnp.float32),
            pltpu.VMEM((b_pad, 8), jnp.float32),
            pltpu.VMEM((_BN, 8), jnp.float32),
        ],
        compiler_params=pltpu.CompilerParams(
            dimension_semantics=("arbitrary",)),
    )(node_feats, segf, degree, g_feats, aw, r, W1.T, W2.T,
      b1.reshape(1, h), b2.reshape(1, h))
    return out


# speedup vs baseline: 3.2075x; 1.0086x over previous
import functools

import jax
import jax.numpy as jnp
import numpy as np
from jax.experimental import pallas as pl
from jax.experimental.pallas import tpu as pltpu

_BN = 4000
_WSZ = 256
_NWIN = 4


def _pool_body(nf_ref, seg_ref, deg_ref, col_ref, g_ref, awr_ref, w1t_ref,
               w2t_ref, b1_ref, b2_ref, out_ref, sd_acc, gz_ref,
               gzv_ref, *, nblocks, b_real):
    i = pl.program_id(0)
    h = nf_ref.shape[1]

    @pl.when(i == 0)
    def _init():
        sd_acc[...] = jnp.zeros_like(sd_acc)
        gz_ref[...] = jnp.zeros_like(gz_ref)
        gz_ref[pl.ds(0, b_real), :] = jax.lax.dot(
            g_ref[...], awr_ref[...], preferred_element_type=jnp.float32)

    nf = nf_ref[...]
    segf = seg_ref[...]
    deg = deg_ref[...]
    col = col_ref[...]
    bn = nf.shape[0]

    zraw = jax.lax.dot(nf, awr_ref[...],
                       preferred_element_type=jnp.float32)

    s_first = segf[0, 0]
    s_last = segf[bn - 1, 0]
    m_first = jnp.floor(s_first / _WSZ)
    m_last = jnp.floor(s_last / _WSZ)

    gzv_ref[...] = jnp.zeros_like(gzv_ref)
    for w in range(_NWIN):
        m = m_first + w

        @pl.when(m <= m_last)
        def _gather(m=m):
            onehot = (segf - m * _WSZ == col).astype(jnp.float32)
            base = m.astype(jnp.int32) * _WSZ
            gzv_ref[...] += jax.lax.dot(
                onehot, gz_ref[pl.ds(base, _WSZ), :],
                preferred_element_type=jnp.float32)

    w128 = jnp.exp((zraw + gzv_ref[...]) * deg)
    u2 = jnp.concatenate([nf * w128, w128], axis=1)

    for w in range(_NWIN):
        m = m_first + w

        @pl.when(m <= m_last)
        def _scatter(m=m):
            onehot = (segf - m * _WSZ == col).astype(jnp.float32)
            base = m.astype(jnp.int32) * _WSZ
            sd_acc[pl.ds(base, _WSZ), :] += jax.lax.dot_general(
                onehot, u2, (((0,), (0,)), ((), ())),
                preferred_element_type=jnp.float32)

    @pl.when(i == nblocks - 1)
    def _finish():
        s = sd_acc[pl.ds(0, b_real), pl.ds(0, h)]
        d = sd_acc[pl.ds(0, b_real), pl.ds(h, h)]
        g = g_ref[...]
        he = jnp.where(d > 0.0, s / d + g, 0.0)
        h1 = jax.nn.relu(
            jax.lax.dot(he, w1t_ref[...], preferred_element_type=jnp.float32)
            + b1_ref[...])
        h2 = jax.lax.dot(h1, w2t_ref[...],
                         preferred_element_type=jnp.float32) + b2_ref[...]
        out_ref[...] = h2 + g


def kernel(node_feats, g_feats, degree, segment_ids, attn, W1, b1, W2, b2):
    n, h = node_feats.shape
    b, _ = g_feats.shape
    nh, dh = attn.shape[1], attn.shape[2]

    segf = segment_ids.astype(jnp.float32).reshape(n, 1)
    col = jnp.arange(_WSZ, dtype=jnp.float32).reshape(1, _WSZ)

    headmask = np.kron(np.eye(nh, dtype=np.float32),
                       np.ones((dh, dh), np.float32))
    awr = attn.reshape(nh * dh, 1) * headmask

    nblocks = n // _BN
    b_pad = _NWIN * _WSZ

    body = functools.partial(_pool_body, nblocks=nblocks, b_real=b)
    out = pl.pallas_call(
        body,
        grid=(nblocks,),
        in_specs=[
            pl.BlockSpec((_BN, h), lambda i: (i, 0)),
            pl.BlockSpec((_BN, 1), lambda i: (i, 0)),
            pl.BlockSpec((_BN, 1), lambda i: (i, 0)),
            pl.BlockSpec((1, _WSZ), lambda i: (0, 0)),
            pl.BlockSpec((b, h), lambda i: (0, 0)),
            pl.BlockSpec((h, h), lambda i: (0, 0)),
            pl.BlockSpec((h, h), lambda i: (0, 0)),
            pl.BlockSpec((h, h), lambda i: (0, 0)),
            pl.BlockSpec((1, h), lambda i: (0, 0)),
            pl.BlockSpec((1, h), lambda i: (0, 0)),
        ],
        out_specs=pl.BlockSpec((b, h), lambda i: (0, 0)),
        out_shape=jax.ShapeDtypeStruct((b, h), jnp.float32),
        scratch_shapes=[
            pltpu.VMEM((b_pad, 2 * h), jnp.float32),
            pltpu.VMEM((b_pad, h), jnp.float32),
            pltpu.VMEM((_BN, h), jnp.float32),
        ],
        compiler_params=pltpu.CompilerParams(
            dimension_semantics=("arbitrary",)),
    )(node_feats, segf, degree, col, g_feats, awr, W1.T, W2.T,
      b1.reshape(1, h), b2.reshape(1, h))
    return out


# speedup vs baseline: 4.2026x; 1.3102x over previous
import functools

import jax
import jax.numpy as jnp
import numpy as np
from jax.experimental import pallas as pl
from jax.experimental.pallas import tpu as pltpu

_BN = 4000
_WSZ = 128
_NWIN = 9
_ROWS = 1152


def _pool_body(nf_ref, seg_ref, deg_ref, col_ref, g_ref, awr_ref, w1t_ref,
               w2t_ref, b1_ref, b2_ref, out_ref, sd_acc, gz_ref, w_ref,
               *, nblocks, b_real):
    i = pl.program_id(0)
    h = nf_ref.shape[1]

    @pl.when(i == 0)
    def _init():
        sd_acc[...] = jnp.zeros_like(sd_acc)
        gz_ref[...] = jnp.zeros_like(gz_ref)
        gz_ref[pl.ds(0, b_real), :] = jax.lax.dot(
            g_ref[...].astype(jnp.bfloat16), awr_ref[...],
            preferred_element_type=jnp.float32)

    nf = nf_ref[...]
    nf_bf = nf.astype(jnp.bfloat16)
    segf = seg_ref[...]
    deg = deg_ref[...]
    col = col_ref[...]
    bn = nf.shape[0]

    zraw = jax.lax.dot(nf_bf, awr_ref[...],
                       preferred_element_type=jnp.float32)

    s_first = segf[0, 0]
    s_last = segf[bn - 1, 0]
    s_base = jnp.floor(s_first / 8.0) * 8.0
    s0i = pl.multiple_of(s_base.astype(jnp.int32), 8)

    def onehot_for(w):
        diff = (segf - (s_base + w * _WSZ)).astype(jnp.bfloat16)
        return (diff == col).astype(jnp.bfloat16)

    onehot0 = onehot_for(0)
    gzv = jax.lax.dot(onehot0, gz_ref[pl.ds(s0i, _WSZ), :],
                      preferred_element_type=jnp.float32)
    w_ref[...] = jnp.exp((zraw + gzv) * deg).astype(jnp.bfloat16)

    for w in range(1, _NWIN):
        @pl.when(s_base + w * _WSZ <= s_last)
        def _extra_gather(w=w):
            ohw = onehot_for(w)
            gzw = jax.lax.dot(ohw, gz_ref[pl.ds(s0i + w * _WSZ, _WSZ), :],
                              preferred_element_type=jnp.float32)
            w_ref[...] *= jnp.exp(gzw * deg).astype(jnp.bfloat16)

    w128 = w_ref[...]
    u2 = jnp.concatenate([nf_bf * w128, w128], axis=1)

    contract = (((0,), (0,)), ((), ()))
    sd_acc[pl.ds(s0i, _WSZ), :] += jax.lax.dot_general(
        onehot0, u2, contract, preferred_element_type=jnp.float32)

    for w in range(1, _NWIN):
        @pl.when(s_base + w * _WSZ <= s_last)
        def _extra_scatter(w=w):
            ohw = onehot_for(w)
            sd_acc[pl.ds(s0i + w * _WSZ, _WSZ), :] += jax.lax.dot_general(
                ohw, u2, contract, preferred_element_type=jnp.float32)

    @pl.when(i == nblocks - 1)
    def _finish():
        s = sd_acc[pl.ds(0, b_real), pl.ds(0, h)]
        d = sd_acc[pl.ds(0, b_real), pl.ds(h, h)]
        g = g_ref[...]
        he = jnp.where(d > 0.0, s / d + g, 0.0)
        h1 = jax.nn.relu(
            jax.lax.dot(he, w1t_ref[...], preferred_element_type=jnp.float32)
            + b1_ref[...])
        h2 = jax.lax.dot(h1, w2t_ref[...],
                         preferred_element_type=jnp.float32) + b2_ref[...]
        out_ref[...] = h2 + g


def kernel(node_feats, g_feats, degree, segment_ids, attn, W1, b1, W2, b2):
    n, h = node_feats.shape
    b, _ = g_feats.shape
    nh, dh = attn.shape[1], attn.shape[2]

    segf = segment_ids.astype(jnp.float32).reshape(n, 1)
    col = jnp.arange(_WSZ, dtype=jnp.float32).reshape(1, _WSZ).astype(
        jnp.bfloat16)

    headmask = np.kron(np.eye(nh, dtype=np.float32),
                       np.ones((dh, dh), np.float32))
    awr = (attn.reshape(nh * dh, 1) * headmask).astype(jnp.bfloat16)

    nblocks = n // _BN

    body = functools.partial(_pool_body, nblocks=nblocks, b_real=b)
    out = pl.pallas_call(
        body,
        grid=(nblocks,),
        in_specs=[
            pl.BlockSpec((_BN, h), lambda i: (i, 0)),
            pl.BlockSpec((_BN, 1), lambda i: (i, 0)),
            pl.BlockSpec((_BN, 1), lambda i: (i, 0)),
            pl.BlockSpec((1, _WSZ), lambda i: (0, 0)),
            pl.BlockSpec((b, h), lambda i: (0, 0)),
            pl.BlockSpec((h, h), lambda i: (0, 0)),
            pl.BlockSpec((h, h), lambda i: (0, 0)),
            pl.BlockSpec((h, h), lambda i: (0, 0)),
            pl.BlockSpec((1, h), lambda i: (0, 0)),
            pl.BlockSpec((1, h), lambda i: (0, 0)),
        ],
        out_specs=pl.BlockSpec((b, h), lambda i: (0, 0)),
        out_shape=jax.ShapeDtypeStruct((b, h), jnp.float32),
        scratch_shapes=[
            pltpu.VMEM((_ROWS, 2 * h), jnp.float32),
            pltpu.VMEM((_ROWS, h), jnp.float32),
            pltpu.VMEM((_BN, h), jnp.bfloat16),
        ],
        compiler_params=pltpu.CompilerParams(
            dimension_semantics=("arbitrary",)),
    )(node_feats, segf, degree, col, g_feats, awr, W1.T, W2.T,
      b1.reshape(1, h), b2.reshape(1, h))
    return out
